# Initial kernel scaffold; baseline (speedup 1.0000x reference)
import jax
import jax.numpy as jnp
from jax.experimental import pallas as pl


def kernel(sequence, emb_weight, W_ih, W_hh, b_ih, b_hh):
    raise NotImplementedError("write your pallas kernel here")


# speedup vs baseline: 1.0974x; 1.0974x over previous
import functools

import jax
import jax.numpy as jnp
from jax import lax
from jax.experimental import pallas as pl
from jax.experimental.pallas import tpu as pltpu
from jax.experimental.pallas import tpu_sc as plsc

D = 64
B = 4096
L = 50

_NC = 2
_NS = 16
_NW = _NC * _NS
_PER_W = (B * L) // _NW
_CH = 800
_NCH = _PER_W // _CH


def _gather_sc(idx, table):
    mesh = plsc.VectorSubcoreMesh(core_axis_name="c", subcore_axis_name="s")

    @functools.partial(
        pl.kernel,
        mesh=mesh,
        out_type=jax.ShapeDtypeStruct((B * L, D), jnp.float32),
        scratch_types=[
            pltpu.VMEM((_NCH, _CH), jnp.int32),
            pltpu.VMEM((_CH, D), jnp.float32),
            pltpu.VMEM((_CH, D), jnp.float32),
            pltpu.SemaphoreType.DMA,
            pltpu.SemaphoreType.DMA,
        ],
        compiler_params=pltpu.CompilerParams(use_tc_tiling_on_sc=False),
    )
    def gather_kernel(idx_hbm, table_hbm, out_hbm, idx_v, buf0, buf1, sem0, sem1):
        wid = lax.axis_index("s") * _NC + lax.axis_index("c")
        base = wid * _PER_W
        pltpu.sync_copy(idx_hbm.at[wid], idx_v)
        bufs = (buf0, buf1)
        sems = (sem0, sem1)
        copies = [None, None]
        copies[0] = pltpu.async_copy(table_hbm.at[idx_v.at[0]], buf0, sem0)
        for c in range(1, _NCH):
            copies[c % 2] = pltpu.async_copy(
                table_hbm.at[idx_v.at[c]], bufs[c % 2], sems[c % 2])
            copies[(c - 1) % 2].wait()
            pltpu.sync_copy(bufs[(c - 1) % 2],
                            out_hbm.at[pl.ds(base + (c - 1) * _CH, _CH)])
        copies[(_NCH - 1) % 2].wait()
        pltpu.sync_copy(bufs[(_NCH - 1) % 2],
                        out_hbm.at[pl.ds(base + (_NCH - 1) * _CH, _CH)])

    return gather_kernel(idx, table)


def _gru_body(x_ref, wih_ref, whh_ref, bih_ref, bhh_ref, out_ref, h_ref):
    @pl.when(pl.program_id(0) == 0)
    def _():
        h_ref[...] = jnp.zeros_like(h_ref)

    wih = wih_ref[...]
    whh = whh_ref[...]
    bih = bih_ref[...]
    bhh = bhh_ref[...]
    h = h_ref[...]
    for t in range(2):
        xt = x_ref[:, t * D:(t + 1) * D]
        gi = jnp.dot(xt, wih, preferred_element_type=jnp.float32) + bih
        gh = jnp.dot(h, whh, preferred_element_type=jnp.float32) + bhh
        r = jax.nn.sigmoid(gi[:, :D] + gh[:, :D])
        z = jax.nn.sigmoid(gi[:, D:2 * D] + gh[:, D:2 * D])
        n = jnp.tanh(gi[:, 2 * D:] + r * gh[:, 2 * D:])
        h = (1.0 - z) * n + z * h
        out_ref[:, t * D:(t + 1) * D] = jnp.maximum(h, 0.0)
    h_ref[...] = h


def _gru_tc(x2, wih_t, whh_t, bih2, bhh2):
    return pl.pallas_call(
        _gru_body,
        grid=(L // 2,),
        in_specs=[
            pl.BlockSpec((B, 2 * D), lambda l: (0, l)),
            pl.BlockSpec((D, 3 * D), lambda l: (0, 0)),
            pl.BlockSpec((D, 3 * D), lambda l: (0, 0)),
            pl.BlockSpec((1, 3 * D), lambda l: (0, 0)),
            pl.BlockSpec((1, 3 * D), lambda l: (0, 0)),
        ],
        out_specs=pl.BlockSpec((B, 2 * D), lambda l: (0, l)),
        out_shape=jax.ShapeDtypeStruct((B, L * D), jnp.float32),
        scratch_shapes=[pltpu.VMEM((B, D), jnp.float32)],
    )(x2, wih_t, whh_t, bih2, bhh2)


def kernel(sequence, emb_weight, W_ih, W_hh, b_ih, b_hh):
    idx = sequence.reshape(_NW, _NCH, _CH)
    x_flat = _gather_sc(idx, emb_weight)
    x2 = x_flat.reshape(B, L * D)
    y = _gru_tc(x2, W_ih.T, W_hh.T,
                b_ih.reshape(1, 3 * D), b_hh.reshape(1, 3 * D))
    return y.reshape(B, L, D)


# speedup vs baseline: 1.2824x; 1.1686x over previous
import functools

import jax
import jax.numpy as jnp
from jax import lax
from jax.experimental import pallas as pl
from jax.experimental.pallas import tpu as pltpu
from jax.experimental.pallas import tpu_sc as plsc

D = 64
B = 4096
L = 50

_NC = 2
_NS = 16
_NW = _NC * _NS
_PER_W = (B * L) // _NW
_CH = 800
_NCH = _PER_W // _CH


def _gather_sc(idx, table):
    mesh = plsc.VectorSubcoreMesh(core_axis_name="c", subcore_axis_name="s")

    @functools.partial(
        pl.kernel,
        mesh=mesh,
        out_type=jax.ShapeDtypeStruct((B * L, D), jnp.float32),
        scratch_types=[
            pltpu.VMEM((_NCH, _CH), jnp.int32),
            pltpu.VMEM((_CH, D), jnp.float32),
            pltpu.VMEM((_CH, D), jnp.float32),
            pltpu.SemaphoreType.DMA,
            pltpu.SemaphoreType.DMA,
        ],
        compiler_params=pltpu.CompilerParams(use_tc_tiling_on_sc=False),
    )
    def gather_kernel(idx_hbm, table_hbm, out_hbm, idx_v, buf0, buf1, sem0, sem1):
        wid = lax.axis_index("s") * _NC + lax.axis_index("c")
        base = wid * _PER_W
        pltpu.sync_copy(idx_hbm.at[wid], idx_v)
        bufs = (buf0, buf1)
        sems = (sem0, sem1)
        copies = [None, None]
        copies[0] = pltpu.async_copy(table_hbm.at[idx_v.at[0]], buf0, sem0)
        for c in range(1, _NCH):
            copies[c % 2] = pltpu.async_copy(
                table_hbm.at[idx_v.at[c]], bufs[c % 2], sems[c % 2])
            copies[(c - 1) % 2].wait()
            pltpu.sync_copy(bufs[(c - 1) % 2],
                            out_hbm.at[pl.ds(base + (c - 1) * _CH, _CH)])
        copies[(_NCH - 1) % 2].wait()
        pltpu.sync_copy(bufs[(_NCH - 1) % 2],
                        out_hbm.at[pl.ds(base + (_NCH - 1) * _CH, _CH)])

    return gather_kernel(idx, table)


_TSTEP = 2
_BH = B // 2


def _gru_body(x_ref, wih_ref, whh_ref, bih_ref, bhh_ref, out_ref,
              he_ref, ho_ref):
    @pl.when(pl.program_id(0) == 0)
    def _():
        he_ref[...] = jnp.zeros_like(he_ref)
        ho_ref[...] = jnp.zeros_like(ho_ref)

    wih = wih_ref[...]
    whh = whh_ref[...]
    bih = bih_ref[...]
    bhh = bhh_ref[...]

    def step(h, xh):
        gi = jax.lax.dot_general(
            wih, xh, (((1,), (1,)), ((), ())),
            preferred_element_type=jnp.float32) + bih
        gh = jnp.dot(whh, h, preferred_element_type=jnp.float32) + bhh
        r = jax.nn.sigmoid(gi[:D] + gh[:D])
        z = jax.nn.sigmoid(gi[D:2 * D] + gh[D:2 * D])
        n = jnp.tanh(gi[2 * D:] + r * gh[2 * D:])
        return (1.0 - z) * n + z * h

    he = he_ref[...]
    ho = ho_ref[...]
    for t in range(_TSTEP):
        xt2 = x_ref[t]
        he = step(he, xt2[:, :D])
        ho = step(ho, xt2[:, D:])
        out_ref[t, :, :_BH] = jnp.maximum(he, 0.0)
        out_ref[t, :, _BH:] = jnp.maximum(ho, 0.0)
    he_ref[...] = he
    ho_ref[...] = ho


def _gru_tc(xf, wih, whh, bih2, bhh2):
    return pl.pallas_call(
        _gru_body,
        grid=(L // _TSTEP,),
        in_specs=[
            pl.BlockSpec((_TSTEP, _BH, 2 * D), lambda l: (l, 0, 0)),
            pl.BlockSpec((3 * D, D), lambda l: (0, 0)),
            pl.BlockSpec((3 * D, D), lambda l: (0, 0)),
            pl.BlockSpec((3 * D, 1), lambda l: (0, 0)),
            pl.BlockSpec((3 * D, 1), lambda l: (0, 0)),
        ],
        out_specs=pl.BlockSpec((_TSTEP, D, B), lambda l: (l, 0, 0)),
        out_shape=jax.ShapeDtypeStruct((L, D, B), jnp.float32),
        scratch_shapes=[pltpu.VMEM((D, _BH), jnp.float32),
                        pltpu.VMEM((D, _BH), jnp.float32)],
    )(xf, wih, whh, bih2, bhh2)


def kernel(sequence, emb_weight, W_ih, W_hh, b_ih, b_hh):
    idx3 = sequence.T.reshape(L, 2, _BH).transpose(0, 2, 1)
    idx = idx3.reshape(_NW, _NCH, _CH)
    x_flat = _gather_sc(idx, emb_weight)
    xf = x_flat.reshape(L, _BH, 2 * D)
    y = _gru_tc(xf, W_ih, W_hh,
                b_ih.reshape(3 * D, 1), b_hh.reshape(3 * D, 1))
    return y.transpose(2, 0, 1)
